# Initial kernel scaffold; baseline (speedup 1.0000x reference)
import jax
import jax.numpy as jnp
from jax.experimental import pallas as pl


def kernel(pos, batch, g1_w, g1_b, l1_w, n1_w, n1_b, a1, g2_w, g2_b, l2_w, n2_w, n2_b, a2, m1_w, m1_b, mn_w, mn_b, ma, m2_w, m2_b):
    raise NotImplementedError("write your pallas kernel here")


# speedup vs baseline: 3.6827x; 3.6827x over previous
import functools

import jax
import jax.numpy as jnp
from jax.experimental import pallas as pl
from jax.experimental.pallas import tpu as pltpu

B = 16
N = 32768
IN = 128
HID = 256
MH = HID // 2
OUT = 64
NEG = -1e30
EPS = 1e-5


def _seg_partial_max(bcol, x, cur):
    parts = [
        jnp.max(jnp.where(bcol == s, x, NEG), axis=0, keepdims=True)
        for s in range(B)
    ]
    return jnp.maximum(cur, jnp.concatenate(parts, axis=0))


def _ln_prelu(h, nw, nb, a):
    mu = jnp.mean(h, axis=1, keepdims=True)
    c = h - mu
    var = jnp.mean(c * c, axis=1, keepdims=True)
    y = c * jax.lax.rsqrt(var + EPS) * nw + nb
    return jnp.where(y >= 0, y, a * y)


def _dot_t(x, w):
    return jax.lax.dot_general(
        x, w, (((1,), (1,)), ((), ())), preferred_element_type=jnp.float32
    )


def _segmax_pos_kernel(pos_ref, b_ref, smax_ref):
    i = pl.program_id(0)

    @pl.when(i == 0)
    def _():
        smax_ref[...] = jnp.full((B, IN), NEG, jnp.float32)

    smax_ref[...] = _seg_partial_max(b_ref[0], pos_ref[...], smax_ref[...])


def _phi_kernel(x_ref, b_ref, smax_ref, lw_ref, gw_ref, gb_ref, nw_ref,
                nb_ref, a_ref, y_ref, smax_out_ref):
    i = pl.program_id(0)

    @pl.when(i == 0)
    def _():
        smax_out_ref[...] = jnp.full(smax_out_ref.shape, NEG, jnp.float32)

    xm = _dot_t(smax_ref[...], lw_ref[...])
    bcol = b_ref[0]
    mask = (bcol == jax.lax.broadcasted_iota(
        jnp.int32, (bcol.shape[0], B), 1)).astype(jnp.float32)
    gath = jax.lax.dot_general(
        mask, xm, (((1,), (0,)), ((), ())), preferred_element_type=jnp.float32)
    h = _dot_t(x_ref[...], gw_ref[...]) + gb_ref[...] - gath
    y = _ln_prelu(h, nw_ref[...], nb_ref[...], a_ref[0, 0])
    y_ref[...] = y
    smax_out_ref[...] = _seg_partial_max(bcol, y, smax_out_ref[...])


def _phi2_head_kernel(x_ref, b_ref, smax_ref, lw_ref, gw_ref, gb_ref, nw_ref,
                      nb_ref, a_ref, m1w_ref, m1b_ref, mnw_ref, mnb_ref,
                      ma_ref, m2w_ref, m2b_ref, out_ref, ssum_ref, scnt_ref):
    i = pl.program_id(0)
    nblk = pl.num_programs(0)

    @pl.when(i == 0)
    def _():
        ssum_ref[...] = jnp.zeros((B, HID), jnp.float32)
        scnt_ref[...] = jnp.zeros((B, HID), jnp.float32)

    xm = _dot_t(smax_ref[...], lw_ref[...])
    bcol = b_ref[0]
    r = bcol.shape[0]
    mask = (bcol == jax.lax.broadcasted_iota(
        jnp.int32, (r, B), 1)).astype(jnp.float32)
    gath = jax.lax.dot_general(
        mask, xm, (((1,), (0,)), ((), ())), preferred_element_type=jnp.float32)
    h = _dot_t(x_ref[...], gw_ref[...]) + gb_ref[...] - gath
    y = _ln_prelu(h, nw_ref[...], nb_ref[...], a_ref[0, 0])
    ssum_ref[...] += jax.lax.dot_general(
        mask, y, (((0,), (0,)), ((), ())), preferred_element_type=jnp.float32)
    scnt_ref[...] += jax.lax.dot_general(
        mask, jnp.ones((r, HID), jnp.float32), (((0,), (0,)), ((), ())),
        preferred_element_type=jnp.float32)

    @pl.when(i == nblk - 1)
    def _():
        pooled = ssum_ref[...] / jnp.maximum(scnt_ref[...], 1.0)
        hh = _dot_t(pooled, m1w_ref[...]) + m1b_ref[...]
        hh = _ln_prelu(hh, mnw_ref[...], mnb_ref[...], ma_ref[0, 0])
        out_ref[...] = _dot_t(hh, m2w_ref[...]) + m2b_ref[...]


def _row(v):
    return v.reshape(1, -1)


def kernel(pos, batch, g1_w, g1_b, l1_w, n1_w, n1_b, a1, g2_w, g2_b, l2_w,
           n2_w, n2_b, a2, m1_w, m1_b, mn_w, mn_b, ma, m2_w, m2_b):
    rb1 = 4096
    nb1 = N // rb1
    rb = 2048
    nb = N // rb
    batch = batch.astype(jnp.int32)
    b3a = batch.reshape(nb1, rb1, 1)
    b3b = batch.reshape(nb, rb, 1)

    full = lambda a: pl.BlockSpec(a.shape, lambda i: (0,) * a.ndim)

    smax0 = pl.pallas_call(
        _segmax_pos_kernel,
        grid=(nb1,),
        in_specs=[
            pl.BlockSpec((rb1, IN), lambda i: (i, 0)),
            pl.BlockSpec((1, rb1, 1), lambda i: (i, 0, 0)),
        ],
        out_specs=pl.BlockSpec((B, IN), lambda i: (0, 0)),
        out_shape=jax.ShapeDtypeStruct((B, IN), jnp.float32),
    )(pos, b3a)

    a1r = a1.reshape(1, 1)
    x1, smax1 = pl.pallas_call(
        _phi_kernel,
        grid=(nb,),
        in_specs=[
            pl.BlockSpec((rb, IN), lambda i: (i, 0)),
            pl.BlockSpec((1, rb, 1), lambda i: (i, 0, 0)),
            full(smax0), full(l1_w), full(g1_w), full(_row(g1_b)),
            full(_row(n1_w)), full(_row(n1_b)), full(a1r),
        ],
        out_specs=[
            pl.BlockSpec((rb, HID), lambda i: (i, 0)),
            pl.BlockSpec((B, HID), lambda i: (0, 0)),
        ],
        out_shape=[
            jax.ShapeDtypeStruct((N, HID), jnp.float32),
            jax.ShapeDtypeStruct((B, HID), jnp.float32),
        ],
    )(pos, b3b, smax0, l1_w, g1_w, _row(g1_b), _row(n1_w), _row(n1_b), a1r)

    a2r = a2.reshape(1, 1)
    mar = ma.reshape(1, 1)
    out = pl.pallas_call(
        _phi2_head_kernel,
        grid=(nb,),
        in_specs=[
            pl.BlockSpec((rb, HID), lambda i: (i, 0)),
            pl.BlockSpec((1, rb, 1), lambda i: (i, 0, 0)),
            full(smax1), full(l2_w), full(g2_w), full(_row(g2_b)),
            full(_row(n2_w)), full(_row(n2_b)), full(a2r),
            full(m1_w), full(_row(m1_b)), full(_row(mn_w)), full(_row(mn_b)),
            full(mar), full(m2_w), full(_row(m2_b)),
        ],
        out_specs=pl.BlockSpec((B, OUT), lambda i: (0, 0)),
        out_shape=jax.ShapeDtypeStruct((B, OUT), jnp.float32),
        scratch_shapes=[
            pltpu.VMEM((B, HID), jnp.float32),
            pltpu.VMEM((B, HID), jnp.float32),
        ],
    )(x1, b3b, smax1, l2_w, g2_w, _row(g2_b), _row(n2_w), _row(n2_b), a2r,
      m1_w, _row(m1_b), _row(mn_w), _row(mn_b), mar, m2_w, _row(m2_b))
    return out


# speedup vs baseline: 6.9889x; 1.8978x over previous
import functools

import jax
import jax.numpy as jnp
from jax.experimental import pallas as pl
from jax.experimental.pallas import tpu as pltpu

B = 16
N = 32768
IN = 128
HID = 256
MH = HID // 2
OUT = 64
NEG = -1e30
EPS = 1e-5


def _seg_partial_max(bcol, x, ref):
    bmin = jnp.min(bcol)
    bmax = jnp.max(bcol)
    for s in range(B):
        @pl.when((bmin <= s) & (s <= bmax))
        def _(s=s):
            m = jnp.max(jnp.where(bcol == s, x, NEG), axis=0, keepdims=True)
            ref[s:s + 1, :] = jnp.maximum(ref[s:s + 1, :], m)


def _ln_prelu(h, nw, nb, a):
    mu = jnp.mean(h, axis=1, keepdims=True)
    c = h - mu
    var = jnp.mean(c * c, axis=1, keepdims=True)
    y = c * jax.lax.rsqrt(var + EPS) * nw + nb
    return jnp.where(y >= 0, y, a * y)


def _dot_t(x, w):
    return jax.lax.dot_general(
        x, w, (((1,), (1,)), ((), ())), preferred_element_type=jnp.float32
    )


def _segmax_pos_kernel(pos_ref, b_ref, smax_ref):
    i = pl.program_id(0)

    @pl.when(i == 0)
    def _():
        smax_ref[...] = jnp.full((B, IN), NEG, jnp.float32)

    _seg_partial_max(b_ref[0], pos_ref[...], smax_ref)


def _phi_kernel(x_ref, b_ref, smax_ref, lw_ref, gw_ref, gb_ref, nw_ref,
                nb_ref, a_ref, y_ref, smax_out_ref):
    i = pl.program_id(0)

    @pl.when(i == 0)
    def _():
        smax_out_ref[...] = jnp.full(smax_out_ref.shape, NEG, jnp.float32)

    xm = _dot_t(smax_ref[...], lw_ref[...])
    bcol = b_ref[0]
    mask = (bcol == jax.lax.broadcasted_iota(
        jnp.int32, (bcol.shape[0], B), 1)).astype(jnp.float32)
    gath = jax.lax.dot_general(
        mask, xm, (((1,), (0,)), ((), ())), preferred_element_type=jnp.float32)
    h = _dot_t(x_ref[...], gw_ref[...]) + gb_ref[...] - gath
    y = _ln_prelu(h, nw_ref[...], nb_ref[...], a_ref[0, 0])
    y_ref[...] = y
    _seg_partial_max(bcol, y, smax_out_ref)


def _phi2_head_kernel(x_ref, b_ref, smax_ref, lw_ref, gw_ref, gb_ref, nw_ref,
                      nb_ref, a_ref, m1w_ref, m1b_ref, mnw_ref, mnb_ref,
                      ma_ref, m2w_ref, m2b_ref, out_ref, ssum_ref, scnt_ref):
    i = pl.program_id(0)
    nblk = pl.num_programs(0)

    @pl.when(i == 0)
    def _():
        ssum_ref[...] = jnp.zeros((B, HID), jnp.float32)
        scnt_ref[...] = jnp.zeros((B, HID), jnp.float32)

    xm = _dot_t(smax_ref[...], lw_ref[...])
    bcol = b_ref[0]
    r = bcol.shape[0]
    mask = (bcol == jax.lax.broadcasted_iota(
        jnp.int32, (r, B), 1)).astype(jnp.float32)
    gath = jax.lax.dot_general(
        mask, xm, (((1,), (0,)), ((), ())), preferred_element_type=jnp.float32)
    h = _dot_t(x_ref[...], gw_ref[...]) + gb_ref[...] - gath
    y = _ln_prelu(h, nw_ref[...], nb_ref[...], a_ref[0, 0])
    ssum_ref[...] += jax.lax.dot_general(
        mask, y, (((0,), (0,)), ((), ())), preferred_element_type=jnp.float32)
    scnt_ref[...] += jax.lax.dot_general(
        mask, jnp.ones((r, HID), jnp.float32), (((0,), (0,)), ((), ())),
        preferred_element_type=jnp.float32)

    @pl.when(i == nblk - 1)
    def _():
        pooled = ssum_ref[...] / jnp.maximum(scnt_ref[...], 1.0)
        hh = _dot_t(pooled, m1w_ref[...]) + m1b_ref[...]
        hh = _ln_prelu(hh, mnw_ref[...], mnb_ref[...], ma_ref[0, 0])
        out_ref[...] = _dot_t(hh, m2w_ref[...]) + m2b_ref[...]


def _row(v):
    return v.reshape(1, -1)


def kernel(pos, batch, g1_w, g1_b, l1_w, n1_w, n1_b, a1, g2_w, g2_b, l2_w,
           n2_w, n2_b, a2, m1_w, m1_b, mn_w, mn_b, ma, m2_w, m2_b):
    rb1 = 4096
    nb1 = N // rb1
    rb = 2048
    nb = N // rb
    batch = batch.astype(jnp.int32)
    b3a = batch.reshape(nb1, rb1, 1)
    b3b = batch.reshape(nb, rb, 1)

    full = lambda a: pl.BlockSpec(a.shape, lambda i: (0,) * a.ndim)

    smax0 = pl.pallas_call(
        _segmax_pos_kernel,
        grid=(nb1,),
        in_specs=[
            pl.BlockSpec((rb1, IN), lambda i: (i, 0)),
            pl.BlockSpec((1, rb1, 1), lambda i: (i, 0, 0)),
        ],
        out_specs=pl.BlockSpec((B, IN), lambda i: (0, 0)),
        out_shape=jax.ShapeDtypeStruct((B, IN), jnp.float32),
    )(pos, b3a)

    a1r = a1.reshape(1, 1)
    x1, smax1 = pl.pallas_call(
        _phi_kernel,
        grid=(nb,),
        in_specs=[
            pl.BlockSpec((rb, IN), lambda i: (i, 0)),
            pl.BlockSpec((1, rb, 1), lambda i: (i, 0, 0)),
            full(smax0), full(l1_w), full(g1_w), full(_row(g1_b)),
            full(_row(n1_w)), full(_row(n1_b)), full(a1r),
        ],
        out_specs=[
            pl.BlockSpec((rb, HID), lambda i: (i, 0)),
            pl.BlockSpec((B, HID), lambda i: (0, 0)),
        ],
        out_shape=[
            jax.ShapeDtypeStruct((N, HID), jnp.float32),
            jax.ShapeDtypeStruct((B, HID), jnp.float32),
        ],
    )(pos, b3b, smax0, l1_w, g1_w, _row(g1_b), _row(n1_w), _row(n1_b), a1r)

    a2r = a2.reshape(1, 1)
    mar = ma.reshape(1, 1)
    out = pl.pallas_call(
        _phi2_head_kernel,
        grid=(nb,),
        in_specs=[
            pl.BlockSpec((rb, HID), lambda i: (i, 0)),
            pl.BlockSpec((1, rb, 1), lambda i: (i, 0, 0)),
            full(smax1), full(l2_w), full(g2_w), full(_row(g2_b)),
            full(_row(n2_w)), full(_row(n2_b)), full(a2r),
            full(m1_w), full(_row(m1_b)), full(_row(mn_w)), full(_row(mn_b)),
            full(mar), full(m2_w), full(_row(m2_b)),
        ],
        out_specs=pl.BlockSpec((B, OUT), lambda i: (0, 0)),
        out_shape=jax.ShapeDtypeStruct((B, OUT), jnp.float32),
        scratch_shapes=[
            pltpu.VMEM((B, HID), jnp.float32),
            pltpu.VMEM((B, HID), jnp.float32),
        ],
    )(x1, b3b, smax1, l2_w, g2_w, _row(g2_b), _row(n2_w), _row(n2_b), a2r,
      m1_w, _row(m1_b), _row(mn_w), _row(mn_b), mar, m2_w, _row(m2_b))
    return out


# speedup vs baseline: 7.8808x; 1.1276x over previous
import jax
import jax.numpy as jnp
from jax.experimental import pallas as pl
from jax.experimental.pallas import tpu as pltpu

B = 16
N = 32768
IN = 128
HID = 256
MH = HID // 2
OUT = 64
RB = 2048
NB = N // RB
NEG = -1e30
EPS = 1e-5


def _seg_partial_max(bcol, x, ref):
    bmin = jnp.min(bcol)
    bmax = jnp.max(bcol)
    for s in range(B):
        @pl.when((bmin <= s) & (s <= bmax))
        def _(s=s):
            m = jnp.max(jnp.where(bcol == s, x, NEG), axis=0, keepdims=True)
            ref[s:s + 1, :] = jnp.maximum(ref[s:s + 1, :], m)


def _ln_prelu(h, nw, nb, a):
    mu = jnp.mean(h, axis=1, keepdims=True)
    c = h - mu
    var = jnp.mean(c * c, axis=1, keepdims=True)
    y = c * jax.lax.rsqrt(var + EPS) * nw + nb
    return jnp.where(y >= 0, y, a * y)


def _dot_t(x, w):
    return jax.lax.dot_general(
        x, w, (((1,), (1,)), ((), ())), preferred_element_type=jnp.float32
    )


def _onehot(bcol):
    return (bcol == jax.lax.broadcasted_iota(
        jnp.int32, (bcol.shape[0], B), 1)).astype(jnp.float32)


def _deepset_kernel(pos_ref, b_ref, l1w_ref, g1w_ref, g1b_ref, n1w_ref,
                    n1b_ref, a1_ref, l2w_ref, g2w_ref, g2b_ref, n2w_ref,
                    n2b_ref, a2_ref, m1w_ref, m1b_ref, mnw_ref, mnb_ref,
                    ma_ref, m2w_ref, m2b_ref, out_ref,
                    smax0_s, x1_s, smax1_s, ssum_s, scnt_s):
    i = pl.program_id(0)

    @pl.when(i == 0)
    def _():
        smax0_s[...] = jnp.full((B, IN), NEG, jnp.float32)
        smax1_s[...] = jnp.full((B, HID), NEG, jnp.float32)
        ssum_s[...] = jnp.zeros((B, HID), jnp.float32)
        scnt_s[...] = jnp.zeros((B, HID), jnp.float32)

    bcol = b_ref[0]
    j = i % NB
    row = pl.multiple_of(j * RB, RB)

    @pl.when(i < NB)
    def _phase_a():
        x1_s[pl.ds(row, RB), :] = _dot_t(pos_ref[...], g1w_ref[...])
        _seg_partial_max(bcol, pos_ref[...], smax0_s)

    @pl.when((i >= NB) & (i < 2 * NB))
    def _phase_b():
        xm = _dot_t(smax0_s[...], l1w_ref[...])
        gath = jax.lax.dot_general(
            _onehot(bcol), xm, (((1,), (0,)), ((), ())),
            preferred_element_type=jnp.float32)
        h = x1_s[pl.ds(row, RB), :] + g1b_ref[...] - gath
        y = _ln_prelu(h, n1w_ref[...], n1b_ref[...], a1_ref[0, 0])
        x1_s[pl.ds(row, RB), :] = y
        _seg_partial_max(bcol, y, smax1_s)

    @pl.when(i >= 2 * NB)
    def _phase_c():
        mask = _onehot(bcol)
        xm = _dot_t(smax1_s[...], l2w_ref[...])
        gath = jax.lax.dot_general(
            mask, xm, (((1,), (0,)), ((), ())),
            preferred_element_type=jnp.float32)
        x = x1_s[pl.ds(row, RB), :]
        h = _dot_t(x, g2w_ref[...]) + g2b_ref[...] - gath
        y = _ln_prelu(h, n2w_ref[...], n2b_ref[...], a2_ref[0, 0])
        ssum_s[...] += jax.lax.dot_general(
            mask, y, (((0,), (0,)), ((), ())),
            preferred_element_type=jnp.float32)
        scnt_s[...] += jax.lax.dot_general(
            mask, jnp.ones((RB, HID), jnp.float32), (((0,), (0,)), ((), ())),
            preferred_element_type=jnp.float32)

        @pl.when(i == 3 * NB - 1)
        def _head():
            pooled = ssum_s[...] / jnp.maximum(scnt_s[...], 1.0)
            hh = _dot_t(pooled, m1w_ref[...]) + m1b_ref[...]
            hh = _ln_prelu(hh, mnw_ref[...], mnb_ref[...], ma_ref[0, 0])
            out_ref[...] = _dot_t(hh, m2w_ref[...]) + m2b_ref[...]


def _row(v):
    return v.reshape(1, -1)


def kernel(pos, batch, g1_w, g1_b, l1_w, n1_w, n1_b, a1, g2_w, g2_b, l2_w,
           n2_w, n2_b, a2, m1_w, m1_b, mn_w, mn_b, ma, m2_w, m2_b):
    batch = batch.astype(jnp.int32)
    b3 = batch.reshape(NB, RB, 1)
    a1r, a2r, mar = a1.reshape(1, 1), a2.reshape(1, 1), ma.reshape(1, 1)

    full = lambda a: pl.BlockSpec(a.shape, lambda i: (0,) * a.ndim)

    def pos_idx(i):
        return (jnp.minimum(i, NB - 1), 0)

    out = pl.pallas_call(
        _deepset_kernel,
        grid=(3 * NB,),
        in_specs=[
            pl.BlockSpec((RB, IN), pos_idx),
            pl.BlockSpec((1, RB, 1), lambda i: (i % NB, 0, 0)),
            full(l1_w), full(g1_w), full(_row(g1_b)), full(_row(n1_w)),
            full(_row(n1_b)), full(a1r),
            full(l2_w), full(g2_w), full(_row(g2_b)), full(_row(n2_w)),
            full(_row(n2_b)), full(a2r),
            full(m1_w), full(_row(m1_b)), full(_row(mn_w)), full(_row(mn_b)),
            full(mar), full(m2_w), full(_row(m2_b)),
        ],
        out_specs=pl.BlockSpec((B, OUT), lambda i: (0, 0)),
        out_shape=jax.ShapeDtypeStruct((B, OUT), jnp.float32),
        scratch_shapes=[
            pltpu.VMEM((B, IN), jnp.float32),
            pltpu.VMEM((N, HID), jnp.float32),
            pltpu.VMEM((B, HID), jnp.float32),
            pltpu.VMEM((B, HID), jnp.float32),
            pltpu.VMEM((B, HID), jnp.float32),
        ],
    )(pos, b3, l1_w, g1_w, _row(g1_b), _row(n1_w), _row(n1_b), a1r,
      l2_w, g2_w, _row(g2_b), _row(n2_w), _row(n2_b), a2r,
      m1_w, _row(m1_b), _row(mn_w), _row(mn_b), mar, m2_w, _row(m2_b))
    return out
